# baseline (speedup 1.0000x reference)
import functools

import jax
import jax.numpy as jnp
from jax import lax
from jax.experimental import pallas as pl
from jax.experimental.pallas import tpu as pltpu
from jax.experimental.pallas import tpu_sc as plsc

N, E, D1, D2, H, L = 10000, 320000, 128, 128, 256, 32
NP = 10240
NC, NS, NW = 2, 16, 32
BE = 80
RB = 400
RPT = NP // NS

f32 = jnp.float32
i32 = jnp.int32

_SC_PARAMS = pltpu.CompilerParams(needs_layout_passes=False)


def _iota16():
    return lax.broadcasted_iota(i32, (16,), 0)


def _proj_body(x_ref, wl_ref, wr_ref, outlf_ref, outrf_ref, outlc_ref):
    x = x_ref[...]
    xl = jnp.dot(x, wl_ref[...], preferred_element_type=f32)
    xr = jnp.dot(x, wr_ref[...], preferred_element_type=f32)
    outlf_ref[...] = xl
    outrf_ref[...] = xr
    outlc_ref[0] = xl[:, :128]
    outlc_ref[1] = xl[:, 128:]


def _proj(x, wl, wr):
    out = pl.pallas_call(
        _proj_body,
        grid=(N // RB,),
        in_specs=[
            pl.BlockSpec((RB, D1), lambda i: (i, 0)),
            pl.BlockSpec((D1, H), lambda i: (0, 0)),
            pl.BlockSpec((D1, H), lambda i: (0, 0)),
        ],
        out_specs=[
            pl.BlockSpec((RB, H), lambda i: (i, 0)),
            pl.BlockSpec((RB, H), lambda i: (i, 0)),
            pl.BlockSpec((2, RB, 128), lambda i: (0, i, 0)),
        ],
        out_shape=[
            jax.ShapeDtypeStruct((N, H), f32),
            jax.ShapeDtypeStruct((N, H), f32),
            jax.ShapeDtypeStruct((2, N, 128), f32),
        ],
    )(x, wl, wr)
    return out[0], out[1], out[2].reshape(2 * N, 128)


def _mid_body(acc_ref, den_ref, b1_ref, wl2_ref, wr2_ref, outl_ref, outr_ref):
    lo = acc_ref[0]
    hi = acc_ref[1]
    den = den_ref[0][:, 0:1] + den_ref[1][:, 0:1] + 1e-16
    h = jnp.concatenate([lo, hi], axis=1) / den + b1_ref[...]
    h = jnp.maximum(h, 0.0)
    xl2 = jnp.dot(h, wl2_ref[...], preferred_element_type=f32)
    xr2 = jnp.dot(h, wr2_ref[...], preferred_element_type=f32)
    pad = jnp.zeros((RB, 128 - 2 * L), f32)
    outl_ref[...] = jnp.concatenate([xl2, pad], axis=1)
    outr_ref[...] = jnp.concatenate([xr2, pad], axis=1)


def _mid(acc1, den1, b1, wl2, wr2):
    return pl.pallas_call(
        _mid_body,
        grid=(N // RB,),
        in_specs=[
            pl.BlockSpec((2, RB, 128), lambda i: (0, i, 0)),
            pl.BlockSpec((2, RB, 128), lambda i: (0, i, 0)),
            pl.BlockSpec((1, H), lambda i: (0, 0)),
            pl.BlockSpec((H, 2 * L), lambda i: (0, 0)),
            pl.BlockSpec((H, 2 * L), lambda i: (0, 0)),
        ],
        out_specs=[
            pl.BlockSpec((RB, 128), lambda i: (i, 0)),
            pl.BlockSpec((RB, 128), lambda i: (i, 0)),
        ],
        out_shape=[
            jax.ShapeDtypeStruct((N, 128), f32),
            jax.ShapeDtypeStruct((N, 128), f32),
        ],
    )(acc1, den1, b1.reshape(1, H), wl2, wr2)


def _fin_body(acc_ref, b2_ref, eps_ref, wm_ref, wlogv_ref, epsw_ref,
              mean_ref, stats_ref):
    a0 = acc_ref[0]
    a1 = acc_ref[1]
    den = a0[:, 64:65] + a1[:, 64:65] + 1e-16
    stats = (a0[:, :64] + a1[:, :64]) / den + b2_ref[...]
    mu = stats[:, :L]
    logvar = stats[:, L:]
    z = mu + eps_ref[...] * jnp.exp(0.5 * logvar)
    wlin = wm_ref[...] + epsw_ref[...] * jnp.exp(0.5 * wlogv_ref[...])
    wmax = jnp.max(wlin, axis=1, keepdims=True)
    we = jnp.exp(wlin - wmax)
    w = we / jnp.sum(we, axis=1, keepdims=True)
    mean_ref[...] = jnp.dot(z, w, preferred_element_type=f32)
    stats_ref[...] = stats


def _fin(acc2, b2, eps, wm, wlogv, epsw):
    D = wm.shape[1]
    return pl.pallas_call(
        _fin_body,
        grid=(N // RB,),
        in_specs=[
            pl.BlockSpec((2, RB, 128), lambda i: (0, i, 0)),
            pl.BlockSpec((1, 2 * L), lambda i: (0, 0)),
            pl.BlockSpec((RB, L), lambda i: (i, 0)),
            pl.BlockSpec((L, D), lambda i: (0, 0)),
            pl.BlockSpec((L, D), lambda i: (0, 0)),
            pl.BlockSpec((L, D), lambda i: (0, 0)),
        ],
        out_specs=[
            pl.BlockSpec((RB, D), lambda i: (i, 0)),
            pl.BlockSpec((RB, 2 * L), lambda i: (i, 0)),
        ],
        out_shape=[
            jax.ShapeDtypeStruct((N, D), f32),
            jax.ShapeDtypeStruct((N, 2 * L), f32),
        ],
    )(acc2, b2.reshape(1, 2 * L), eps, wm, wlogv, epsw)


def _mesh():
    return plsc.VectorSubcoreMesh(core_axis_name="c", subcore_axis_name="s")


def _zero_rows(buf, nrows, wo):
    def zrow(j, carry):
        for k in range(wo // 16):
            buf[j, pl.ds(k * 16, 16)] = jnp.zeros((16,), f32)
        return carry

    lax.fori_loop(0, nrows, zrow, 0)


def _zero_acc(acc_sh, zbuf, s):
    for k in range(RPT // BE):
        pltpu.sync_copy(zbuf, acc_sh.at[pl.ds(s * RPT + k * BE, BE)])


def _writeout(acc_sh, out_hbm, c, s):
    for k in range(RPT // BE):
        r0 = s * RPT + k * BE
        pltpu.sync_copy(acc_sh.at[pl.ds(r0, BE)],
                        out_hbm.at[c, pl.ds(r0, BE)])


def _make_score1():
    ept = E // NW
    nch = ept // BE

    @functools.partial(
        pl.kernel,
        compiler_params=_SC_PARAMS,
        out_type=jax.ShapeDtypeStruct((E,), f32),
        mesh=_mesh(),
        scratch_types=[
            pltpu.VMEM((ept,), i32),
            pltpu.VMEM((ept,), i32),
            pltpu.VMEM((BE, H), f32),
            pltpu.VMEM((BE, H), f32),
            pltpu.VMEM((BE, H), f32),
            pltpu.VMEM((BE, H), f32),
            pltpu.VMEM((ept,), f32),
            pltpu.VMEM((H * 16,), f32),
            pltpu.SemaphoreType.DMA,
            pltpu.SemaphoreType.DMA,
        ],
    )
    def score1(xl_hbm, xr_hbm, src_hbm, dst_hbm, att_hbm, ex_hbm,
               srcb, dstb, bl0, br0, bl1, br1, exb, attv, sem0, sem1):
        wid = lax.axis_index("s") * NC + lax.axis_index("c")
        base = wid * ept
        pltpu.sync_copy(att_hbm, attv)
        pltpu.sync_copy(src_hbm.at[pl.ds(base, ept)], srcb)
        pltpu.sync_copy(dst_hbm.at[pl.ds(base, ept)], dstb)
        it16 = _iota16()
        sets = ((bl0, br0, sem0), (bl1, br1, sem1))

        def issue(ci, bl, br, sem):
            sl = pl.ds(ci * BE, BE)
            pltpu.async_copy(xl_hbm.at[srcb.at[sl]], bl, sem)
            pltpu.async_copy(xr_hbm.at[dstb.at[sl]], br, sem)

        def drain(ci, bl, br, sem):
            sl = pl.ds(ci * BE, BE)
            pltpu.make_async_copy(xl_hbm.at[srcb.at[sl]], bl, sem).wait()
            pltpu.make_async_copy(xr_hbm.at[dstb.at[sl]], br, sem).wait()

        def compute(ci, bl, br):
            for g in range(BE // 16):
                rows = it16 + g * 16

                def hbody(hh, acc):
                    col = jnp.zeros((16,), i32) + hh
                    vl = plsc.load_gather(bl, [rows, col])
                    vr = plsc.load_gather(br, [rows, col])
                    u = vl + vr
                    u = jnp.maximum(u, 0.2 * u)
                    return acc + u * attv[pl.ds(hh * 16, 16)]

                acc = lax.fori_loop(0, H, hbody, jnp.zeros((16,), f32),
                                    unroll=8)
                exb[pl.ds(ci * BE + g * 16, 16)] = jnp.exp(acc)

        issue(0, *sets[0])
        issue(1, *sets[1])

        def step(i2, carry):
            for par in range(2):
                ci = 2 * i2 + par
                drain(ci, *sets[par])
                compute(ci, *sets[par][:2])

                @pl.when(ci + 2 < nch)
                def _():
                    issue(ci + 2, *sets[par])
            return carry

        lax.fori_loop(0, (nch - 1) // 2, step, 0)
        ci = nch - 1
        drain(ci, *sets[0])
        compute(ci, *sets[0][:2])
        pltpu.sync_copy(exb, ex_hbm.at[pl.ds(base, ept)])

    return score1


def _make_den1():
    ept = E // NW
    SB = 5 * BE

    @functools.partial(
        pl.kernel,
        compiler_params=_SC_PARAMS,
        out_type=jax.ShapeDtypeStruct((2, NP, 128), f32),
        mesh=_mesh(),
        scratch_types=[
            pltpu.VMEM((SB,), i32),
            pltpu.VMEM((SB,), f32),
            pltpu.VMEM((BE, 128), f32),
            pltpu.VMEM((BE,), i32),
            pltpu.VMEM_SHARED((NP, 128), f32),
            pltpu.SemaphoreType.DMA,
        ],
    )
    def den1(dst_hbm, ex_hbm, out_hbm, dstb, exb, scb, dstv, acc_sh, sem):
        c = lax.axis_index("c")
        s = lax.axis_index("s")
        base = (s * NC + c) * ept
        it16 = _iota16()
        _zero_rows(scb, BE, 128)
        _zero_acc(acc_sh, scb, s)
        plsc.subcore_barrier()

        def block(bi, carry):
            bo = base + bi * SB
            pltpu.sync_copy(dst_hbm.at[pl.ds(bo, SB)], dstb)
            pltpu.sync_copy(ex_hbm.at[pl.ds(bo, SB)], exb)

            def chunk(cj, carry2):
                for g in range(BE // 16):
                    r16 = it16 + g * 16
                    sl = pl.ds(cj * BE + g * 16, 16)
                    dstv[pl.ds(g * 16, 16)] = dstb[sl]
                    plsc.store_scatter(scb, [r16, jnp.zeros((16,), i32)],
                                       exb[sl])
                pltpu.sync_copy(scb, acc_sh.at[dstv], add=True)
                return carry2

            lax.fori_loop(0, SB // BE, chunk, 0)
            return carry

        lax.fori_loop(0, ept // SB, block, 0)
        plsc.subcore_barrier()
        _writeout(acc_sh, out_hbm, c, s)

    return den1


def _make_accum1():
    ept = E // NS
    SB = 10 * BE
    nblk = ept // SB

    @functools.partial(
        pl.kernel,
        compiler_params=_SC_PARAMS,
        out_type=jax.ShapeDtypeStruct((2, NP, 128), f32),
        mesh=_mesh(),
        scratch_types=[
            pltpu.VMEM((SB,), i32),
            pltpu.VMEM((SB,), i32),
            pltpu.VMEM((SB,), f32),
            pltpu.VMEM((BE, 128), f32),
            pltpu.VMEM((BE, 128), f32),
            pltpu.VMEM((BE,), i32),
            pltpu.VMEM_SHARED((NP, 128), f32),
            pltpu.SemaphoreType.DMA,
            pltpu.SemaphoreType.DMA,
        ],
    )
    def accum1(xl_hbm, src_hbm, dst_hbm, ex_hbm, out_hbm,
               srcb, dstb, exb, rows0, rows1, dstv, acc_sh, sem0, sem1):
        c = lax.axis_index("c")
        s = lax.axis_index("s")
        base = s * ept
        it16 = _iota16()
        _zero_rows(rows0, BE, 128)
        _zero_acc(acc_sh, rows0, s)
        plsc.subcore_barrier()

        sems = (sem0, sem1)
        rows_sets = (rows0, rows1)

        def issue(ci, rows, sem):
            pltpu.async_copy(xl_hbm.at[srcb.at[pl.ds(ci * BE, BE)]],
                             rows, sem)

        def drain(ci, rows, sem):
            pltpu.make_async_copy(xl_hbm.at[srcb.at[pl.ds(ci * BE, BE)]],
                                  rows, sem).wait()

        def compute(ci, rows):
            for g in range(BE // 16):
                r16 = it16 + g * 16
                sl = pl.ds(ci * BE + g * 16, 16)
                exg = exb[sl]
                dstv[pl.ds(g * 16, 16)] = dstb[sl]

                def hbody(hh, carry2):
                    col = jnp.zeros((16,), i32) + hh
                    v = plsc.load_gather(rows, [r16, col]) * exg
                    plsc.store_scatter(rows, [r16, col], v)
                    return carry2

                lax.fori_loop(0, 128, hbody, 0, unroll=8)
            pltpu.sync_copy(rows, acc_sh.at[dstv], add=True)

        def block(bi, carry):
            bo = base + bi * SB
            pltpu.sync_copy(src_hbm.at[pl.ds(bo, SB)], srcb)
            pltpu.sync_copy(dst_hbm.at[pl.ds(bo, SB)], dstb)
            pltpu.sync_copy(ex_hbm.at[pl.ds(bo, SB)], exb)

            def adj(j, carry2):
                sl = pl.ds(j * 16, 16)
                srcb[sl] = srcb[sl] + c * N
                return carry2

            lax.fori_loop(0, SB // 16, adj, 0)
            issue(0, rows0, sem0)
            issue(1, rows1, sem1)

            def step(i2, carry2):
                for par in range(2):
                    ci = 2 * i2 + par
                    drain(ci, rows_sets[par], sems[par])
                    compute(ci, rows_sets[par])

                    @pl.when(ci + 2 < SB // BE)
                    def _():
                        issue(ci + 2, rows_sets[par], sems[par])
                return carry2

            lax.fori_loop(0, SB // BE // 2, step, 0)
            return carry

        lax.fori_loop(0, nblk, block, 0)
        plsc.subcore_barrier()
        _writeout(acc_sh, out_hbm, c, s)

    return accum1


def _make_gat2():
    wf = 2 * L
    ept = E // NW
    SB = 10 * BE
    nblk = ept // SB

    @functools.partial(
        pl.kernel,
        compiler_params=_SC_PARAMS,
        out_type=jax.ShapeDtypeStruct((2, NP, 128), f32),
        mesh=_mesh(),
        scratch_types=[
            pltpu.VMEM((5 * BE,), i32),
            pltpu.VMEM((5 * BE,), i32),
            pltpu.VMEM((BE, 128), f32),
            pltpu.VMEM((BE, 128), f32),
            pltpu.VMEM((BE, 128), f32),
            pltpu.VMEM((BE, 128), f32),
            pltpu.VMEM((wf * 16,), f32),
            pltpu.VMEM((BE,), i32),
            pltpu.VMEM_SHARED((NP, 128), f32),
            pltpu.SemaphoreType.DMA,
            pltpu.SemaphoreType.DMA,
        ],
    )
    def gat2(xl_hbm, xr_hbm, src_hbm, dst_hbm, att_hbm, out_hbm,
             srcb, dstb, bl0, br0, bl1, br1, attv, dstv, acc_sh,
             sem0, sem1):
        SBn = 5 * BE
        c = lax.axis_index("c")
        s = lax.axis_index("s")
        base = (s * NC + c) * ept
        pltpu.sync_copy(att_hbm, attv)
        it16 = _iota16()
        _zero_rows(bl0, BE, 128)
        _zero_acc(acc_sh, bl0, s)
        plsc.subcore_barrier()

        sets = ((bl0, br0, sem0), (bl1, br1, sem1))

        def issue(ci, bl, br, sem):
            sl = pl.ds(ci * BE, BE)
            pltpu.async_copy(xl_hbm.at[srcb.at[sl]], bl, sem)
            pltpu.async_copy(xr_hbm.at[dstb.at[sl]], br, sem)

        def drain(ci, bl, br, sem):
            sl = pl.ds(ci * BE, BE)
            pltpu.make_async_copy(xl_hbm.at[srcb.at[sl]], bl, sem).wait()
            pltpu.make_async_copy(xr_hbm.at[dstb.at[sl]], br, sem).wait()

        def compute(ci, bl, br):
            for g in range(BE // 16):
                r16 = it16 + g * 16
                dstv[pl.ds(g * 16, 16)] = dstb[pl.ds(ci * BE + g * 16, 16)]

                def hbody(hh, acc):
                    col = jnp.zeros((16,), i32) + hh
                    vl = plsc.load_gather(bl, [r16, col])
                    vr = plsc.load_gather(br, [r16, col])
                    u = vl + vr
                    u = jnp.maximum(u, 0.2 * u)
                    return acc + u * attv[pl.ds(hh * 16, 16)]

                acc = lax.fori_loop(0, wf, hbody, jnp.zeros((16,), f32),
                                    unroll=8)
                exg = jnp.exp(acc)
                plsc.store_scatter(bl, [r16, jnp.zeros((16,), i32) + wf],
                                   exg)

                def sbody(hh, carry2):
                    col = jnp.zeros((16,), i32) + hh
                    v = plsc.load_gather(bl, [r16, col]) * exg
                    plsc.store_scatter(bl, [r16, col], v)
                    return carry2

                lax.fori_loop(0, wf, sbody, 0, unroll=8)
            pltpu.sync_copy(bl, acc_sh.at[dstv], add=True)

        def block(bi, carry):
            bo = base + bi * SBn
            pltpu.sync_copy(src_hbm.at[pl.ds(bo, SBn)], srcb)
            pltpu.sync_copy(dst_hbm.at[pl.ds(bo, SBn)], dstb)
            issue(0, *sets[0])
            issue(1, *sets[1])

            def step(i2, carry2):
                for par in range(2):
                    ci = 2 * i2 + par
                    drain(ci, *sets[par])
                    compute(ci, *sets[par][:2])

                    @pl.when(ci + 2 < 5)
                    def _():
                        issue(ci + 2, *sets[par])
                return carry2

            lax.fori_loop(0, 2, step, 0)
            ci = 4
            drain(ci, *sets[0])
            compute(ci, *sets[0][:2])
            return carry

        lax.fori_loop(0, ept // SBn, block, 0)
        plsc.subcore_barrier()
        _writeout(acc_sh, out_hbm, c, s)

    return gat2


def kernel(x1, x2, edge_index,
           enc1_Wl1, enc1_Wr1, enc1_a1, enc1_b1,
           enc1_Wl2, enc1_Wr2, enc1_a2, enc1_b2,
           enc2_Wl1, enc2_Wr1, enc2_a1, enc2_b1,
           enc2_Wl2, enc2_Wr2, enc2_a2, enc2_b2,
           dec1_w_m, dec1_w_logv, dec2_w_m, dec2_w_logv):
    src, dst = edge_index[0], edge_index[1]
    rk = jax.random.key(1)
    eps1 = jax.random.normal(jax.random.fold_in(rk, 0), (N, L), f32)
    eps2 = jax.random.normal(jax.random.fold_in(rk, 1), (N, L), f32)
    epsw1 = jax.random.normal(jax.random.fold_in(rk, 2), (L, D1), f32)
    epsw2 = jax.random.normal(jax.random.fold_in(rk, 3), (L, D2), f32)

    score1 = _make_score1()
    den1 = _make_den1()
    accum1 = _make_accum1()
    gat2 = _make_gat2()

    def encoder(x, wl1, wr1, a1, b1, wl2, wr2, a2, b2):
        xlf, xrf, xlc = _proj(x, wl1, wr1)
        a1sp = jnp.broadcast_to(a1[:, None], (H, 16)).reshape(H * 16)
        a2sp = jnp.broadcast_to(a2[:, None], (2 * L, 16)).reshape(2 * L * 16)
        ex1 = score1(xlf, xrf, src, dst, a1sp)
        d1 = den1(dst, ex1)
        acc1 = accum1(xlc, src, dst, ex1)
        xl2, xr2 = _mid(acc1, d1, b1, wl2, wr2)
        acc2 = gat2(xl2, xr2, src, dst, a2sp)
        return acc2

    acc2_1 = encoder(x1, enc1_Wl1, enc1_Wr1, enc1_a1, enc1_b1,
                     enc1_Wl2, enc1_Wr2, enc1_a2, enc1_b2)
    acc2_2 = encoder(x2, enc2_Wl1, enc2_Wr1, enc2_a1, enc2_b1,
                     enc2_Wl2, enc2_Wr2, enc2_a2, enc2_b2)

    mean1, stats1 = _fin(acc2_1, enc1_b2, eps1, dec1_w_m, dec1_w_logv, epsw1)
    mean2, stats2 = _fin(acc2_2, enc2_b2, eps2, dec2_w_m, dec2_w_logv, epsw2)

    return (mean1, mean2, stats1[:, :L], stats1[:, L:],
            stats2[:, :L], stats2[:, L:])


# speedup vs baseline: 2.1881x; 2.1881x over previous
import functools

import jax
import jax.numpy as jnp
from jax import lax
from jax.experimental import pallas as pl
from jax.experimental.pallas import tpu as pltpu
from jax.experimental.pallas import tpu_sc as plsc

N, E, D1, D2, H, L = 10000, 320000, 128, 128, 256, 32
NC, NS, NW = 2, 16, 32
BE = 80
RB = 400
RPT = N // NS

f32 = jnp.float32
i32 = jnp.int32
bf16 = jnp.bfloat16
HP = H // 2

_SC_PARAMS = pltpu.CompilerParams(
    needs_layout_passes=False, use_tc_tiling_on_sc=False)


def _iota16():
    return lax.broadcasted_iota(i32, (16,), 0)


def _unpack2(w):
    lo = lax.bitcast_convert_type(lax.shift_left(w, 16), f32)
    hi = lax.bitcast_convert_type(jnp.bitwise_and(w, jnp.int32(-65536)), f32)
    return lo, hi


def _pack_rows(a, rows, feats):
    return lax.bitcast_convert_type(
        a.reshape(rows, feats // 2, 2).astype(bf16), i32)


def _proj_body(x_ref, wl_ref, wr_ref, outlf_ref, outrf_ref, outlc_ref):
    x = x_ref[...]
    xl = jnp.dot(x, wl_ref[...], preferred_element_type=f32)
    xr = jnp.dot(x, wr_ref[...], preferred_element_type=f32)
    xlb = xl.astype(bf16)
    outlf_ref[...] = xlb
    outrf_ref[...] = xr.astype(bf16)
    outlc_ref[0] = xlb[:, :128]
    outlc_ref[1] = xlb[:, 128:]


def _proj(x, wl, wr):
    out = pl.pallas_call(
        _proj_body,
        grid=(N // RB,),
        in_specs=[
            pl.BlockSpec((RB, D1), lambda i: (i, 0)),
            pl.BlockSpec((D1, H), lambda i: (0, 0)),
            pl.BlockSpec((D1, H), lambda i: (0, 0)),
        ],
        out_specs=[
            pl.BlockSpec((RB, H), lambda i: (i, 0)),
            pl.BlockSpec((RB, H), lambda i: (i, 0)),
            pl.BlockSpec((2, RB, 128), lambda i: (0, i, 0)),
        ],
        out_shape=[
            jax.ShapeDtypeStruct((N, H), bf16),
            jax.ShapeDtypeStruct((N, H), bf16),
            jax.ShapeDtypeStruct((2, N, 128), bf16),
        ],
    )(x, wl, wr)
    return (_pack_rows(out[0], N, H), _pack_rows(out[1], N, H),
            _pack_rows(out[2].reshape(2 * N, 128), 2 * N, 128))


def _mid_body(acc_ref, b1_ref, wl2_ref, wr2_ref, outl_ref, outr_ref):
    lo = acc_ref[0]
    hi = acc_ref[1]
    den = lo[:, 128:129] + 1e-16
    h = jnp.concatenate([lo[:, :128], hi[:, :128]], axis=1) / den + b1_ref[...]
    h = jnp.maximum(h, 0.0)
    outl_ref[...] = jnp.dot(h, wl2_ref[...],
                            preferred_element_type=f32).astype(bf16)
    outr_ref[...] = jnp.dot(h, wr2_ref[...],
                            preferred_element_type=f32).astype(bf16)


def _mid(acc1, b1, wl2, wr2):
    return pl.pallas_call(
        _mid_body,
        grid=(N // RB,),
        in_specs=[
            pl.BlockSpec((2, RB, 144), lambda i: (0, i, 0)),
            pl.BlockSpec((1, H), lambda i: (0, 0)),
            pl.BlockSpec((H, 2 * L), lambda i: (0, 0)),
            pl.BlockSpec((H, 2 * L), lambda i: (0, 0)),
        ],
        out_specs=[
            pl.BlockSpec((RB, 2 * L), lambda i: (i, 0)),
            pl.BlockSpec((RB, 2 * L), lambda i: (i, 0)),
        ],
        out_shape=[
            jax.ShapeDtypeStruct((N, 2 * L), bf16),
            jax.ShapeDtypeStruct((N, 2 * L), bf16),
        ],
    )(acc1, b1.reshape(1, H), wl2, wr2)


def _fin_body(acc_ref, b2_ref, eps_ref, wm_ref, wlogv_ref, epsw_ref,
              mean_ref, stats_ref):
    a0 = acc_ref[0]
    a1 = acc_ref[1]
    den = a0[:, 64:65] + a1[:, 64:65] + 1e-16
    stats = (a0[:, :64] + a1[:, :64]) / den + b2_ref[...]
    mu = stats[:, :L]
    logvar = stats[:, L:]
    z = mu + eps_ref[...] * jnp.exp(0.5 * logvar)
    wlin = wm_ref[...] + epsw_ref[...] * jnp.exp(0.5 * wlogv_ref[...])
    wmax = jnp.max(wlin, axis=1, keepdims=True)
    we = jnp.exp(wlin - wmax)
    w = we / jnp.sum(we, axis=1, keepdims=True)
    mean_ref[...] = jnp.dot(z, w, preferred_element_type=f32)
    stats_ref[...] = stats


def _fin(acc2, b2, eps, wm, wlogv, epsw):
    D = wm.shape[1]
    return pl.pallas_call(
        _fin_body,
        grid=(N // RB,),
        in_specs=[
            pl.BlockSpec((2, RB, 80), lambda i: (0, i, 0)),
            pl.BlockSpec((1, 2 * L), lambda i: (0, 0)),
            pl.BlockSpec((RB, L), lambda i: (i, 0)),
            pl.BlockSpec((L, D), lambda i: (0, 0)),
            pl.BlockSpec((L, D), lambda i: (0, 0)),
            pl.BlockSpec((L, D), lambda i: (0, 0)),
        ],
        out_specs=[
            pl.BlockSpec((RB, D), lambda i: (i, 0)),
            pl.BlockSpec((RB, 2 * L), lambda i: (i, 0)),
        ],
        out_shape=[
            jax.ShapeDtypeStruct((N, D), f32),
            jax.ShapeDtypeStruct((N, 2 * L), f32),
        ],
    )(acc2, b2.reshape(1, 2 * L), eps, wm, wlogv, epsw)


def _mesh():
    return plsc.VectorSubcoreMesh(core_axis_name="c", subcore_axis_name="s")


def _zero_rows(buf, nrows, wo):
    def zrow(j, carry):
        for k in range(wo // 16):
            buf[j, pl.ds(k * 16, 16)] = jnp.zeros((16,), f32)
        return carry

    lax.fori_loop(0, nrows, zrow, 0)


def _zero_acc(acc_sh, zbuf, s):
    for k in range(RPT // BE):
        pltpu.sync_copy(zbuf, acc_sh.at[pl.ds(s * RPT + k * BE, BE)])
    rem = RPT % BE
    if rem:
        pltpu.sync_copy(zbuf.at[pl.ds(0, rem)],
                        acc_sh.at[pl.ds(s * RPT + RPT - rem, rem)])


def _writeout(acc_sh, out_hbm, c, s, wo):
    for k in range(RPT // BE):
        r0 = s * RPT + k * BE
        pltpu.sync_copy(acc_sh.at[pl.ds(r0, BE)],
                        out_hbm.at[c, pl.ds(r0, BE)])
    rem = RPT % BE
    if rem:
        r0 = s * RPT + RPT - rem
        pltpu.sync_copy(acc_sh.at[pl.ds(r0, rem)],
                        out_hbm.at[c, pl.ds(r0, rem)])


def _make_score1():
    ept = E // NW
    nch = ept // BE

    @functools.partial(
        pl.kernel,
        compiler_params=_SC_PARAMS,
        out_type=jax.ShapeDtypeStruct((E,), f32),
        mesh=_mesh(),
        scratch_types=[
            pltpu.VMEM((ept,), i32),
            pltpu.VMEM((ept,), i32),
            pltpu.VMEM((BE, HP), i32),
            pltpu.VMEM((BE, HP), i32),
            pltpu.VMEM((BE, HP), i32),
            pltpu.VMEM((BE, HP), i32),
            pltpu.VMEM((ept,), f32),
            pltpu.VMEM((H, 16), f32),
            pltpu.SemaphoreType.DMA,
            pltpu.SemaphoreType.DMA,
        ],
    )
    def score1(xl_hbm, xr_hbm, src_hbm, dst_hbm, att_hbm, ex_hbm,
               srcb, dstb, bl0, br0, bl1, br1, exb, attv, sem0, sem1):
        wid = lax.axis_index("s") * NC + lax.axis_index("c")
        base = wid * ept
        pltpu.sync_copy(att_hbm, attv)
        pltpu.sync_copy(src_hbm.at[pl.ds(base, ept)], srcb)
        pltpu.sync_copy(dst_hbm.at[pl.ds(base, ept)], dstb)
        it16 = _iota16()
        sets = ((bl0, br0, sem0), (bl1, br1, sem1))

        def issue(ci, bl, br, sem):
            sl = pl.ds(ci * BE, BE)
            pltpu.async_copy(xl_hbm.at[srcb.at[sl]], bl, sem)
            pltpu.async_copy(xr_hbm.at[dstb.at[sl]], br, sem)

        def drain(ci, bl, br, sem):
            sl = pl.ds(ci * BE, BE)
            pltpu.make_async_copy(xl_hbm.at[srcb.at[sl]], bl, sem).wait()
            pltpu.make_async_copy(xr_hbm.at[dstb.at[sl]], br, sem).wait()

        def compute(ci, bl, br):
            for g in range(BE // 16):
                rows = it16 + g * 16

                def hbody(hh, acc):
                    col = jnp.zeros((16,), i32) + hh
                    pwl = plsc.load_gather(bl, [rows, col])
                    pwr = plsc.load_gather(br, [rows, col])
                    l0, l1 = _unpack2(pwl)
                    r0, r1 = _unpack2(pwr)
                    u0 = l0 + r0
                    u0 = jnp.maximum(u0, 0.2 * u0)
                    u1 = l1 + r1
                    u1 = jnp.maximum(u1, 0.2 * u1)
                    return acc + u0 * attv[2 * hh] + u1 * attv[2 * hh + 1]

                acc = lax.fori_loop(0, HP, hbody, jnp.zeros((16,), f32),
                                    unroll=8)
                exb[pl.ds(ci * BE + g * 16, 16)] = jnp.exp(acc)

        issue(0, *sets[0])
        issue(1, *sets[1])

        def step(i2, carry):
            for par in range(2):
                ci = 2 * i2 + par
                drain(ci, *sets[par])
                compute(ci, *sets[par][:2])

                @pl.when(ci + 2 < nch)
                def _():
                    issue(ci + 2, *sets[par])
            return carry

        lax.fori_loop(0, (nch - 1) // 2, step, 0)
        ci = nch - 1
        drain(ci, *sets[0])
        compute(ci, *sets[0][:2])
        pltpu.sync_copy(exb, ex_hbm.at[pl.ds(base, ept)])

    return score1


def _make_accum1():
    wf, wo = 128, 144
    ept = E // NS
    SB = 10 * BE
    nblk = ept // SB

    @functools.partial(
        pl.kernel,
        compiler_params=_SC_PARAMS,
        out_type=jax.ShapeDtypeStruct((2, N, wo), f32),
        mesh=_mesh(),
        scratch_types=[
            pltpu.VMEM((SB,), i32),
            pltpu.VMEM((SB,), i32),
            pltpu.VMEM((SB,), f32),
            pltpu.VMEM((BE, wf // 2), i32),
            pltpu.VMEM((BE, wf // 2), i32),
            pltpu.VMEM((BE, wo), f32),
            pltpu.VMEM((BE,), i32),
            pltpu.VMEM_SHARED((N, wo), f32),
            pltpu.SemaphoreType.DMA,
            pltpu.SemaphoreType.DMA,
        ],
    )
    def accum1(xl_hbm, src_hbm, dst_hbm, ex_hbm, out_hbm,
               srcb, dstb, exb, rows0, rows1, scal, dstv,
               acc_sh, sem0, sem1):
        c = lax.axis_index("c")
        s = lax.axis_index("s")
        base = s * ept
        it16 = _iota16()
        _zero_rows(scal, BE, wo)
        _zero_acc(acc_sh, scal, s)
        plsc.subcore_barrier()

        sems = (sem0, sem1)
        rows_sets = (rows0, rows1)

        def issue(ci, rows, sem):
            pltpu.async_copy(xl_hbm.at[srcb.at[pl.ds(ci * BE, BE)]],
                             rows, sem)

        def drain(ci, rows, sem):
            pltpu.make_async_copy(xl_hbm.at[srcb.at[pl.ds(ci * BE, BE)]],
                                  rows, sem).wait()

        def compute(ci, rows):
            for g in range(BE // 16):
                r16 = it16 + g * 16
                sl = pl.ds(ci * BE + g * 16, 16)
                exg = exb[sl]
                dstv[pl.ds(g * 16, 16)] = dstb[sl]
                plsc.store_scatter(scal, [r16, jnp.zeros((16,), i32) + wf],
                                   exg)

                def hbody(hh, carry2):
                    col = jnp.zeros((16,), i32) + hh
                    w = plsc.load_gather(rows, [r16, col])
                    v0, v1 = _unpack2(w)
                    plsc.store_scatter(scal, [r16, 2 * col], v0 * exg)
                    plsc.store_scatter(scal, [r16, 2 * col + 1], v1 * exg)
                    return carry2

                lax.fori_loop(0, wf // 2, hbody, 0, unroll=8)
            pltpu.sync_copy(scal, acc_sh.at[dstv], add=True)

        def block(bi, carry):
            bo = base + bi * SB
            pltpu.sync_copy(src_hbm.at[pl.ds(bo, SB)], srcb)
            pltpu.sync_copy(dst_hbm.at[pl.ds(bo, SB)], dstb)
            pltpu.sync_copy(ex_hbm.at[pl.ds(bo, SB)], exb)

            def adj(j, carry2):
                sl = pl.ds(j * 16, 16)
                srcb[sl] = srcb[sl] + c * N
                return carry2

            lax.fori_loop(0, SB // 16, adj, 0)
            issue(0, rows0, sem0)
            issue(1, rows1, sem1)

            def step(i2, carry2):
                for par in range(2):
                    ci = 2 * i2 + par
                    drain(ci, rows_sets[par], sems[par])
                    compute(ci, rows_sets[par])

                    @pl.when(ci + 2 < SB // BE)
                    def _():
                        issue(ci + 2, rows_sets[par], sems[par])
                return carry2

            lax.fori_loop(0, SB // BE // 2, step, 0)
            return carry

        lax.fori_loop(0, nblk, block, 0)
        plsc.subcore_barrier()
        _writeout(acc_sh, out_hbm, c, s, wo)

    return accum1


def _make_gat2():
    wf, wo = 64, 80
    ept = E // NW
    nch = ept // BE

    @functools.partial(
        pl.kernel,
        compiler_params=_SC_PARAMS,
        out_type=jax.ShapeDtypeStruct((2, N, wo), f32),
        mesh=_mesh(),
        scratch_types=[
            pltpu.VMEM((ept,), i32),
            pltpu.VMEM((ept,), i32),
            pltpu.VMEM((BE, wf // 2), i32),
            pltpu.VMEM((BE, wf // 2), i32),
            pltpu.VMEM((BE, wf // 2), i32),
            pltpu.VMEM((BE, wf // 2), i32),
            pltpu.VMEM((BE, wo), f32),
            pltpu.VMEM((BE, wo), f32),
            pltpu.VMEM((wf, 16), f32),
            pltpu.VMEM((BE,), i32),
            pltpu.VMEM_SHARED((N, wo), f32),
            pltpu.SemaphoreType.DMA,
            pltpu.SemaphoreType.DMA,
        ],
    )
    def gat2(xl_hbm, xr_hbm, src_hbm, dst_hbm, att_hbm, out_hbm,
             srcb, dstb, bl0, br0, bl1, br1, scal0, scal1, attv, dstv,
             acc_sh, sem0, sem1):
        c = lax.axis_index("c")
        s = lax.axis_index("s")
        wid = s * NC + c
        base = wid * ept
        pltpu.sync_copy(att_hbm, attv)
        pltpu.sync_copy(src_hbm.at[pl.ds(base, ept)], srcb)
        pltpu.sync_copy(dst_hbm.at[pl.ds(base, ept)], dstb)
        it16 = _iota16()
        _zero_rows(scal0, BE, wo)
        _zero_rows(scal1, BE, wo)
        _zero_acc(acc_sh, scal0, s)
        plsc.subcore_barrier()

        sets = ((bl0, br0, scal0, sem0), (bl1, br1, scal1, sem1))

        def issue(ci, bl, br, scal, sem):
            sl = pl.ds(ci * BE, BE)
            pltpu.async_copy(xl_hbm.at[srcb.at[sl]], bl, sem)
            pltpu.async_copy(xr_hbm.at[dstb.at[sl]], br, sem)

        def drain(ci, bl, br, scal, sem):
            sl = pl.ds(ci * BE, BE)
            pltpu.make_async_copy(xl_hbm.at[srcb.at[sl]], bl, sem).wait()
            pltpu.make_async_copy(xr_hbm.at[dstb.at[sl]], br, sem).wait()

        def compute(ci, bl, br, scal, sem):
            for g in range(BE // 16):
                r16 = it16 + g * 16
                dstv[pl.ds(g * 16, 16)] = dstb[pl.ds(ci * BE + g * 16, 16)]

                def hbody(hh, acc):
                    col = jnp.zeros((16,), i32) + hh
                    pwl = plsc.load_gather(bl, [r16, col])
                    pwr = plsc.load_gather(br, [r16, col])
                    l0, l1 = _unpack2(pwl)
                    r0, r1 = _unpack2(pwr)
                    u0 = l0 + r0
                    u0 = jnp.maximum(u0, 0.2 * u0)
                    u1 = l1 + r1
                    u1 = jnp.maximum(u1, 0.2 * u1)
                    return acc + u0 * attv[2 * hh] + u1 * attv[2 * hh + 1]

                acc = lax.fori_loop(0, wf // 2, hbody, jnp.zeros((16,), f32),
                                    unroll=8)
                exg = jnp.exp(acc)
                plsc.store_scatter(scal, [r16, jnp.zeros((16,), i32) + wf],
                                   exg)

                def sbody(hh, carry2):
                    col = jnp.zeros((16,), i32) + hh
                    w = plsc.load_gather(bl, [r16, col])
                    v0, v1 = _unpack2(w)
                    plsc.store_scatter(scal, [r16, 2 * col], v0 * exg)
                    plsc.store_scatter(scal, [r16, 2 * col + 1], v1 * exg)
                    return carry2

                lax.fori_loop(0, wf // 2, sbody, 0, unroll=8)
            pltpu.sync_copy(scal, acc_sh.at[dstv], add=True)

        issue(0, *sets[0])
        issue(1, *sets[1])

        def step(i2, carry):
            for par in range(2):
                ci = 2 * i2 + par
                drain(ci, *sets[par])
                compute(ci, *sets[par])

                @pl.when(ci + 2 < nch)
                def _():
                    issue(ci + 2, *sets[par])
            return carry

        lax.fori_loop(0, (nch - 1) // 2, step, 0)
        ci = nch - 1
        drain(ci, *sets[0])
        compute(ci, *sets[0])
        plsc.subcore_barrier()
        _writeout(acc_sh, out_hbm, c, s, wo)

    return gat2


def kernel(x1, x2, edge_index,
           enc1_Wl1, enc1_Wr1, enc1_a1, enc1_b1,
           enc1_Wl2, enc1_Wr2, enc1_a2, enc1_b2,
           enc2_Wl1, enc2_Wr1, enc2_a1, enc2_b1,
           enc2_Wl2, enc2_Wr2, enc2_a2, enc2_b2,
           dec1_w_m, dec1_w_logv, dec2_w_m, dec2_w_logv):
    src, dst = edge_index[0], edge_index[1]
    rk = jax.random.key(1)
    eps1 = jax.random.normal(jax.random.fold_in(rk, 0), (N, L), f32)
    eps2 = jax.random.normal(jax.random.fold_in(rk, 1), (N, L), f32)
    epsw1 = jax.random.normal(jax.random.fold_in(rk, 2), (L, D1), f32)
    epsw2 = jax.random.normal(jax.random.fold_in(rk, 3), (L, D2), f32)

    score1 = _make_score1()
    accum1 = _make_accum1()
    gat2 = _make_gat2()

    def encoder(x, wl1, wr1, a1, b1, wl2, wr2, a2, b2):
        xlf, xrf, xlc = _proj(x, wl1, wr1)
        a1sp = jnp.broadcast_to(a1[:, None], (H, 16))
        a2sp = jnp.broadcast_to(a2[:, None], (2 * L, 16))
        ex1 = score1(xlf, xrf, src, dst, a1sp)
        acc1 = accum1(xlc, src, dst, ex1)
        xl2, xr2 = _mid(acc1, b1, wl2, wr2)
        xl2p = _pack_rows(xl2, N, 2 * L)
        xr2p = _pack_rows(xr2, N, 2 * L)
        acc2 = gat2(xl2p, xr2p, src, dst, a2sp)
        return acc2

    acc2_1 = encoder(x1, enc1_Wl1, enc1_Wr1, enc1_a1, enc1_b1,
                     enc1_Wl2, enc1_Wr2, enc1_a2, enc1_b2)
    acc2_2 = encoder(x2, enc2_Wl1, enc2_Wr1, enc2_a1, enc2_b1,
                     enc2_Wl2, enc2_Wr2, enc2_a2, enc2_b2)

    mean1, stats1 = _fin(acc2_1, enc1_b2, eps1, dec1_w_m, dec1_w_logv, epsw1)
    mean2, stats2 = _fin(acc2_2, enc2_b2, eps2, dec2_w_m, dec2_w_logv, epsw2)

    return (mean1, mean2, stats1[:, :L], stats1[:, L:],
            stats2[:, :L], stats2[:, L:])


# speedup vs baseline: 2.2856x; 1.0446x over previous
import functools

import jax
import jax.numpy as jnp
from jax import lax
from jax.experimental import pallas as pl
from jax.experimental.pallas import tpu as pltpu
from jax.experimental.pallas import tpu_sc as plsc

N, E, D1, D2, H, L = 10000, 320000, 128, 128, 256, 32
NC, NS, NW = 2, 16, 32
BE = 80
RB = 400
RPT = N // NS

f32 = jnp.float32
i32 = jnp.int32
bf16 = jnp.bfloat16
HP = H // 2

_SC_PARAMS = pltpu.CompilerParams(
    needs_layout_passes=False, use_tc_tiling_on_sc=False)


def _iota16():
    return lax.broadcasted_iota(i32, (16,), 0)


def _unpack2(w):
    lo = lax.bitcast_convert_type(lax.shift_left(w, 16), f32)
    hi = lax.bitcast_convert_type(jnp.bitwise_and(w, jnp.int32(-65536)), f32)
    return lo, hi


def _pack_rows(a, rows, feats):
    return lax.bitcast_convert_type(
        a.reshape(rows, feats // 2, 2).astype(bf16), i32)


def _proj_body(x_ref, wl_ref, wr_ref, outlf_ref, outrf_ref, outlc_ref):
    x = x_ref[...]
    xl = jnp.dot(x, wl_ref[...], preferred_element_type=f32)
    xr = jnp.dot(x, wr_ref[...], preferred_element_type=f32)
    xlb = xl.astype(bf16)
    outlf_ref[...] = xlb
    outrf_ref[...] = xr.astype(bf16)
    outlc_ref[0] = xlb[:, :128]
    outlc_ref[1] = xlb[:, 128:]


def _proj(x, wl, wr):
    out = pl.pallas_call(
        _proj_body,
        grid=(N // RB,),
        in_specs=[
            pl.BlockSpec((RB, D1), lambda i: (i, 0)),
            pl.BlockSpec((D1, H), lambda i: (0, 0)),
            pl.BlockSpec((D1, H), lambda i: (0, 0)),
        ],
        out_specs=[
            pl.BlockSpec((RB, H), lambda i: (i, 0)),
            pl.BlockSpec((RB, H), lambda i: (i, 0)),
            pl.BlockSpec((2, RB, 128), lambda i: (0, i, 0)),
        ],
        out_shape=[
            jax.ShapeDtypeStruct((N, H), bf16),
            jax.ShapeDtypeStruct((N, H), bf16),
            jax.ShapeDtypeStruct((2, N, 128), bf16),
        ],
    )(x, wl, wr)
    return (_pack_rows(out[0], N, H), _pack_rows(out[1], N, H),
            _pack_rows(out[2].reshape(2 * N, 128), 2 * N, 128))


def _mid_body(acc_ref, b1_ref, wl2_ref, wr2_ref, outl_ref, outr_ref):
    lo = acc_ref[0]
    hi = acc_ref[1]
    den = lo[:, 128:129] + 1e-16
    h = jnp.concatenate([lo[:, :128], hi[:, :128]], axis=1) / den + b1_ref[...]
    h = jnp.maximum(h, 0.0)
    outl_ref[...] = jnp.dot(h, wl2_ref[...],
                            preferred_element_type=f32).astype(bf16)
    outr_ref[...] = jnp.dot(h, wr2_ref[...],
                            preferred_element_type=f32).astype(bf16)


def _mid(acc1, b1, wl2, wr2):
    return pl.pallas_call(
        _mid_body,
        grid=(N // RB,),
        in_specs=[
            pl.BlockSpec((2, RB, 144), lambda i: (0, i, 0)),
            pl.BlockSpec((1, H), lambda i: (0, 0)),
            pl.BlockSpec((H, 2 * L), lambda i: (0, 0)),
            pl.BlockSpec((H, 2 * L), lambda i: (0, 0)),
        ],
        out_specs=[
            pl.BlockSpec((RB, 2 * L), lambda i: (i, 0)),
            pl.BlockSpec((RB, 2 * L), lambda i: (i, 0)),
        ],
        out_shape=[
            jax.ShapeDtypeStruct((N, 2 * L), bf16),
            jax.ShapeDtypeStruct((N, 2 * L), bf16),
        ],
    )(acc1, b1.reshape(1, H), wl2, wr2)


def _fin_body(acc_ref, b2_ref, eps_ref, wm_ref, wlogv_ref, epsw_ref,
              mean_ref, stats_ref):
    a0 = acc_ref[0]
    a1 = acc_ref[1]
    den = a0[:, 64:65] + a1[:, 64:65] + 1e-16
    stats = (a0[:, :64] + a1[:, :64]) / den + b2_ref[...]
    mu = stats[:, :L]
    logvar = stats[:, L:]
    z = mu + eps_ref[...] * jnp.exp(0.5 * logvar)
    wlin = wm_ref[...] + epsw_ref[...] * jnp.exp(0.5 * wlogv_ref[...])
    wmax = jnp.max(wlin, axis=1, keepdims=True)
    we = jnp.exp(wlin - wmax)
    w = we / jnp.sum(we, axis=1, keepdims=True)
    mean_ref[...] = jnp.dot(z, w, preferred_element_type=f32)
    stats_ref[...] = stats


def _fin(acc2, b2, eps, wm, wlogv, epsw):
    D = wm.shape[1]
    return pl.pallas_call(
        _fin_body,
        grid=(N // RB,),
        in_specs=[
            pl.BlockSpec((2, RB, 80), lambda i: (0, i, 0)),
            pl.BlockSpec((1, 2 * L), lambda i: (0, 0)),
            pl.BlockSpec((RB, L), lambda i: (i, 0)),
            pl.BlockSpec((L, D), lambda i: (0, 0)),
            pl.BlockSpec((L, D), lambda i: (0, 0)),
            pl.BlockSpec((L, D), lambda i: (0, 0)),
        ],
        out_specs=[
            pl.BlockSpec((RB, D), lambda i: (i, 0)),
            pl.BlockSpec((RB, 2 * L), lambda i: (i, 0)),
        ],
        out_shape=[
            jax.ShapeDtypeStruct((N, D), f32),
            jax.ShapeDtypeStruct((N, 2 * L), f32),
        ],
    )(acc2, b2.reshape(1, 2 * L), eps, wm, wlogv, epsw)


def _mesh():
    return plsc.VectorSubcoreMesh(core_axis_name="c", subcore_axis_name="s")


def _zero_rows(buf, nrows, wo):
    def zrow(j, carry):
        for k in range(wo // 16):
            buf[j, pl.ds(k * 16, 16)] = jnp.zeros((16,), f32)
        return carry

    lax.fori_loop(0, nrows, zrow, 0)


def _zero_acc(acc_sh, zbuf, s):
    for k in range(RPT // BE):
        pltpu.sync_copy(zbuf, acc_sh.at[pl.ds(s * RPT + k * BE, BE)])
    rem = RPT % BE
    if rem:
        pltpu.sync_copy(zbuf.at[pl.ds(0, rem)],
                        acc_sh.at[pl.ds(s * RPT + RPT - rem, rem)])


def _writeout(acc_sh, out_hbm, c, s, wo):
    for k in range(RPT // BE):
        r0 = s * RPT + k * BE
        pltpu.sync_copy(acc_sh.at[pl.ds(r0, BE)],
                        out_hbm.at[c, pl.ds(r0, BE)])
    rem = RPT % BE
    if rem:
        r0 = s * RPT + RPT - rem
        pltpu.sync_copy(acc_sh.at[pl.ds(r0, rem)],
                        out_hbm.at[c, pl.ds(r0, rem)])


def _make_score1():
    ept = E // NW
    nch = ept // BE

    @functools.partial(
        pl.kernel,
        compiler_params=_SC_PARAMS,
        out_type=jax.ShapeDtypeStruct((E,), f32),
        mesh=_mesh(),
        scratch_types=[
            pltpu.VMEM((ept,), i32),
            pltpu.VMEM((ept,), i32),
            pltpu.VMEM((BE, HP), i32),
            pltpu.VMEM((BE, HP), i32),
            pltpu.VMEM((BE, HP), i32),
            pltpu.VMEM((BE, HP), i32),
            pltpu.VMEM((ept,), f32),
            pltpu.VMEM((H, 16), f32),
            pltpu.SemaphoreType.DMA,
            pltpu.SemaphoreType.DMA,
        ],
    )
    def score1(xl_hbm, xr_hbm, src_hbm, dst_hbm, att_hbm, ex_hbm,
               srcb, dstb, bl0, br0, bl1, br1, exb, attv, sem0, sem1):
        wid = lax.axis_index("s") * NC + lax.axis_index("c")
        base = wid * ept
        pltpu.sync_copy(att_hbm, attv)
        pltpu.sync_copy(src_hbm.at[pl.ds(base, ept)], srcb)
        pltpu.sync_copy(dst_hbm.at[pl.ds(base, ept)], dstb)
        it16 = _iota16()
        sets = ((bl0, br0, sem0), (bl1, br1, sem1))

        def issue(ci, bl, br, sem):
            sl = pl.ds(ci * BE, BE)
            pltpu.async_copy(xl_hbm.at[srcb.at[sl]], bl, sem)
            pltpu.async_copy(xr_hbm.at[dstb.at[sl]], br, sem)

        def drain(ci, bl, br, sem):
            sl = pl.ds(ci * BE, BE)
            pltpu.make_async_copy(xl_hbm.at[srcb.at[sl]], bl, sem).wait()
            pltpu.make_async_copy(xr_hbm.at[dstb.at[sl]], br, sem).wait()

        def compute(ci, bl, br):
            for g in range(BE // 16):
                rows = it16 + g * 16

                def hbody(hh, acc):
                    col = jnp.zeros((16,), i32) + hh
                    pwl = plsc.load_gather(bl, [rows, col])
                    pwr = plsc.load_gather(br, [rows, col])
                    l0, l1 = _unpack2(pwl)
                    r0, r1 = _unpack2(pwr)
                    u0 = l0 + r0
                    u0 = jnp.maximum(u0, 0.2 * u0)
                    u1 = l1 + r1
                    u1 = jnp.maximum(u1, 0.2 * u1)
                    return acc + u0 * attv[2 * hh] + u1 * attv[2 * hh + 1]

                acc = lax.fori_loop(0, HP, hbody, jnp.zeros((16,), f32),
                                    unroll=8)
                exb[pl.ds(ci * BE + g * 16, 16)] = jnp.exp(acc)

        issue(0, *sets[0])
        issue(1, *sets[1])

        def step(i2, carry):
            for par in range(2):
                ci = 2 * i2 + par
                drain(ci, *sets[par])
                compute(ci, *sets[par][:2])

                @pl.when(ci + 2 < nch)
                def _():
                    issue(ci + 2, *sets[par])
            return carry

        lax.fori_loop(0, (nch - 1) // 2, step, 0)
        ci = nch - 1
        drain(ci, *sets[0])
        compute(ci, *sets[0][:2])
        pltpu.sync_copy(exb, ex_hbm.at[pl.ds(base, ept)])

    return score1


def _make_accum1():
    wf, wo = 128, 144
    ept = E // NS
    SB = 10 * BE
    nblk = ept // SB

    @functools.partial(
        pl.kernel,
        compiler_params=_SC_PARAMS,
        out_type=jax.ShapeDtypeStruct((2, N, wo), f32),
        mesh=_mesh(),
        scratch_types=[
            pltpu.VMEM((SB,), i32),
            pltpu.VMEM((SB,), i32),
            pltpu.VMEM((SB,), f32),
            pltpu.VMEM((BE, wf // 2), i32),
            pltpu.VMEM((BE, wf // 2), i32),
            pltpu.VMEM((BE, wo), f32),
            pltpu.VMEM((BE, wo), f32),
            pltpu.VMEM((BE,), i32),
            pltpu.VMEM((BE,), i32),
            pltpu.VMEM_SHARED((N, wo), f32),
            pltpu.SemaphoreType.DMA,
            pltpu.SemaphoreType.DMA,
            pltpu.SemaphoreType.DMA,
            pltpu.SemaphoreType.DMA,
        ],
    )
    def accum1(xl_hbm, src_hbm, dst_hbm, ex_hbm, out_hbm,
               srcb, dstb, exb, rows0, rows1, scal0, scal1, dstv0, dstv1,
               acc_sh, sem0, sem1, ssem0, ssem1):
        c = lax.axis_index("c")
        s = lax.axis_index("s")
        base = s * ept
        it16 = _iota16()
        _zero_rows(scal0, BE, wo)
        _zero_rows(scal1, BE, wo)
        _zero_acc(acc_sh, scal0, s)
        plsc.subcore_barrier()

        sems = (sem0, sem1)
        rows_sets = (rows0, rows1)
        scals = (scal0, scal1)
        dstvs = (dstv0, dstv1)
        ssems = (ssem0, ssem1)

        def issue(ci, rows, sem):
            pltpu.async_copy(xl_hbm.at[srcb.at[pl.ds(ci * BE, BE)]],
                             rows, sem)

        def drain(ci, rows, sem):
            pltpu.make_async_copy(xl_hbm.at[srcb.at[pl.ds(ci * BE, BE)]],
                                  rows, sem).wait()

        def compute(gi, ci, rows, par):
            scal = scals[par]
            dstv = dstvs[par]
            ssem = ssems[par]

            @pl.when(gi >= 2)
            def _():
                pltpu.make_async_copy(scal, acc_sh.at[dstv], ssem).wait()

            for g in range(BE // 16):
                r16 = it16 + g * 16
                sl = pl.ds(ci * BE + g * 16, 16)
                exg = exb[sl]
                dstv[pl.ds(g * 16, 16)] = dstb[sl]
                plsc.store_scatter(scal, [r16, jnp.zeros((16,), i32) + wf],
                                   exg)

                def hbody(hh, carry2):
                    col = jnp.zeros((16,), i32) + hh
                    w = plsc.load_gather(rows, [r16, col])
                    v0, v1 = _unpack2(w)
                    plsc.store_scatter(scal, [r16, 2 * col], v0 * exg)
                    plsc.store_scatter(scal, [r16, 2 * col + 1], v1 * exg)
                    return carry2

                lax.fori_loop(0, wf // 2, hbody, 0, unroll=8)
            pltpu.async_copy(scal, acc_sh.at[dstv], ssem, add=True)

        def block(bi, carry):
            bo = base + bi * SB
            pltpu.sync_copy(src_hbm.at[pl.ds(bo, SB)], srcb)
            pltpu.sync_copy(dst_hbm.at[pl.ds(bo, SB)], dstb)
            pltpu.sync_copy(ex_hbm.at[pl.ds(bo, SB)], exb)

            def adj(j, carry2):
                sl = pl.ds(j * 16, 16)
                srcb[sl] = srcb[sl] + c * N
                return carry2

            lax.fori_loop(0, SB // 16, adj, 0)
            issue(0, rows0, sem0)
            issue(1, rows1, sem1)

            def step(i2, carry2):
                for par in range(2):
                    ci = 2 * i2 + par
                    drain(ci, rows_sets[par], sems[par])
                    compute(bi * (SB // BE) + ci, ci, rows_sets[par], par)

                    @pl.when(ci + 2 < SB // BE)
                    def _():
                        issue(ci + 2, rows_sets[par], sems[par])
                return carry2

            lax.fori_loop(0, SB // BE // 2, step, 0)
            return carry

        lax.fori_loop(0, nblk, block, 0)
        pltpu.make_async_copy(scal0, acc_sh.at[dstv0], ssem0).wait()
        pltpu.make_async_copy(scal1, acc_sh.at[dstv1], ssem1).wait()
        plsc.subcore_barrier()
        _writeout(acc_sh, out_hbm, c, s, wo)

    return accum1


def _make_gat2():
    wf, wo = 64, 80
    ept = E // NW
    nch = ept // BE

    @functools.partial(
        pl.kernel,
        compiler_params=_SC_PARAMS,
        out_type=jax.ShapeDtypeStruct((2, N, wo), f32),
        mesh=_mesh(),
        scratch_types=[
            pltpu.VMEM((ept,), i32),
            pltpu.VMEM((ept,), i32),
            pltpu.VMEM((BE, wf // 2), i32),
            pltpu.VMEM((BE, wf // 2), i32),
            pltpu.VMEM((BE, wf // 2), i32),
            pltpu.VMEM((BE, wf // 2), i32),
            pltpu.VMEM((BE, wo), f32),
            pltpu.VMEM((BE, wo), f32),
            pltpu.VMEM((wf, 16), f32),
            pltpu.VMEM((BE,), i32),
            pltpu.VMEM((BE,), i32),
            pltpu.VMEM_SHARED((N, wo), f32),
            pltpu.SemaphoreType.DMA,
            pltpu.SemaphoreType.DMA,
            pltpu.SemaphoreType.DMA,
            pltpu.SemaphoreType.DMA,
        ],
    )
    def gat2(xl_hbm, xr_hbm, src_hbm, dst_hbm, att_hbm, out_hbm,
             srcb, dstb, bl0, br0, bl1, br1, scal0, scal1, attv,
             dstv0, dstv1, acc_sh, sem0, sem1, ssem0, ssem1):
        c = lax.axis_index("c")
        s = lax.axis_index("s")
        wid = s * NC + c
        base = wid * ept
        pltpu.sync_copy(att_hbm, attv)
        pltpu.sync_copy(src_hbm.at[pl.ds(base, ept)], srcb)
        pltpu.sync_copy(dst_hbm.at[pl.ds(base, ept)], dstb)
        it16 = _iota16()
        _zero_rows(scal0, BE, wo)
        _zero_rows(scal1, BE, wo)
        _zero_acc(acc_sh, scal0, s)
        plsc.subcore_barrier()

        sets = ((bl0, br0, scal0, dstv0, sem0, ssem0),
                (bl1, br1, scal1, dstv1, sem1, ssem1))

        def issue(ci, bl, br, scal, dstv, sem, ssem):
            sl = pl.ds(ci * BE, BE)
            pltpu.async_copy(xl_hbm.at[srcb.at[sl]], bl, sem)
            pltpu.async_copy(xr_hbm.at[dstb.at[sl]], br, sem)

        def drain(ci, bl, br, scal, dstv, sem, ssem):
            sl = pl.ds(ci * BE, BE)
            pltpu.make_async_copy(xl_hbm.at[srcb.at[sl]], bl, sem).wait()
            pltpu.make_async_copy(xr_hbm.at[dstb.at[sl]], br, sem).wait()

        def compute(ci, bl, br, scal, dstv, sem, ssem):
            @pl.when(ci >= 2)
            def _():
                pltpu.make_async_copy(scal, acc_sh.at[dstv], ssem).wait()

            for g in range(BE // 16):
                r16 = it16 + g * 16
                dstv[pl.ds(g * 16, 16)] = dstb[pl.ds(ci * BE + g * 16, 16)]

                def hbody(hh, acc):
                    col = jnp.zeros((16,), i32) + hh
                    pwl = plsc.load_gather(bl, [r16, col])
                    pwr = plsc.load_gather(br, [r16, col])
                    l0, l1 = _unpack2(pwl)
                    r0, r1 = _unpack2(pwr)
                    u0 = l0 + r0
                    u0 = jnp.maximum(u0, 0.2 * u0)
                    u1 = l1 + r1
                    u1 = jnp.maximum(u1, 0.2 * u1)
                    return acc + u0 * attv[2 * hh] + u1 * attv[2 * hh + 1]

                acc = lax.fori_loop(0, wf // 2, hbody, jnp.zeros((16,), f32),
                                    unroll=8)
                exg = jnp.exp(acc)
                plsc.store_scatter(scal, [r16, jnp.zeros((16,), i32) + wf],
                                   exg)

                def sbody(hh, carry2):
                    col = jnp.zeros((16,), i32) + hh
                    w = plsc.load_gather(bl, [r16, col])
                    v0, v1 = _unpack2(w)
                    plsc.store_scatter(scal, [r16, 2 * col], v0 * exg)
                    plsc.store_scatter(scal, [r16, 2 * col + 1], v1 * exg)
                    return carry2

                lax.fori_loop(0, wf // 2, sbody, 0, unroll=8)
            pltpu.async_copy(scal, acc_sh.at[dstv], ssem, add=True)

        issue(0, *sets[0])
        issue(1, *sets[1])

        def step(i2, carry):
            for par in range(2):
                ci = 2 * i2 + par
                drain(ci, *sets[par])
                compute(ci, *sets[par])

                @pl.when(ci + 2 < nch)
                def _():
                    issue(ci + 2, *sets[par])
            return carry

        lax.fori_loop(0, (nch - 1) // 2, step, 0)
        ci = nch - 1
        drain(ci, *sets[0])
        compute(ci, *sets[0])
        pltpu.make_async_copy(scal0, acc_sh.at[dstv0], ssem0).wait()
        pltpu.make_async_copy(scal1, acc_sh.at[dstv1], ssem1).wait()
        plsc.subcore_barrier()
        _writeout(acc_sh, out_hbm, c, s, wo)

    return gat2


def kernel(x1, x2, edge_index,
           enc1_Wl1, enc1_Wr1, enc1_a1, enc1_b1,
           enc1_Wl2, enc1_Wr2, enc1_a2, enc1_b2,
           enc2_Wl1, enc2_Wr1, enc2_a1, enc2_b1,
           enc2_Wl2, enc2_Wr2, enc2_a2, enc2_b2,
           dec1_w_m, dec1_w_logv, dec2_w_m, dec2_w_logv):
    src, dst = edge_index[0], edge_index[1]
    rk = jax.random.key(1)
    eps1 = jax.random.normal(jax.random.fold_in(rk, 0), (N, L), f32)
    eps2 = jax.random.normal(jax.random.fold_in(rk, 1), (N, L), f32)
    epsw1 = jax.random.normal(jax.random.fold_in(rk, 2), (L, D1), f32)
    epsw2 = jax.random.normal(jax.random.fold_in(rk, 3), (L, D2), f32)

    score1 = _make_score1()
    accum1 = _make_accum1()
    gat2 = _make_gat2()

    def encoder(x, wl1, wr1, a1, b1, wl2, wr2, a2, b2):
        xlf, xrf, xlc = _proj(x, wl1, wr1)
        a1sp = jnp.broadcast_to(a1[:, None], (H, 16))
        a2sp = jnp.broadcast_to(a2[:, None], (2 * L, 16))
        ex1 = score1(xlf, xrf, src, dst, a1sp)
        acc1 = accum1(xlc, src, dst, ex1)
        xl2, xr2 = _mid(acc1, b1, wl2, wr2)
        xl2p = _pack_rows(xl2, N, 2 * L)
        xr2p = _pack_rows(xr2, N, 2 * L)
        acc2 = gat2(xl2p, xr2p, src, dst, a2sp)
        return acc2

    acc2_1 = encoder(x1, enc1_Wl1, enc1_Wr1, enc1_a1, enc1_b1,
                     enc1_Wl2, enc1_Wr2, enc1_a2, enc1_b2)
    acc2_2 = encoder(x2, enc2_Wl1, enc2_Wr1, enc2_a1, enc2_b1,
                     enc2_Wl2, enc2_Wr2, enc2_a2, enc2_b2)

    mean1, stats1 = _fin(acc2_1, enc1_b2, eps1, dec1_w_m, dec1_w_logv, epsw1)
    mean2, stats2 = _fin(acc2_2, enc2_b2, eps2, dec2_w_m, dec2_w_logv, epsw2)

    return (mean1, mean2, stats1[:, :L], stats1[:, L:],
            stats2[:, :L], stats2[:, L:])


# speedup vs baseline: 2.2998x; 1.0062x over previous
import functools

import jax
import jax.numpy as jnp
from jax import lax
from jax.experimental import pallas as pl
from jax.experimental.pallas import tpu as pltpu
from jax.experimental.pallas import tpu_sc as plsc

N, E, D1, D2, H, L = 10000, 320000, 128, 128, 256, 32
NC, NS, NW = 2, 16, 32
BE = 80
RB = 400
RPT = N // NS

f32 = jnp.float32
i32 = jnp.int32
bf16 = jnp.bfloat16
HP = H // 2

_SC_PARAMS = pltpu.CompilerParams(
    needs_layout_passes=False, use_tc_tiling_on_sc=False)


def _iota16():
    return lax.broadcasted_iota(i32, (16,), 0)


def _unpack2(w):
    lo = lax.bitcast_convert_type(lax.shift_left(w, 16), f32)
    hi = lax.bitcast_convert_type(jnp.bitwise_and(w, jnp.int32(-65536)), f32)
    return lo, hi


def _pack_rows(a, rows, feats):
    return lax.bitcast_convert_type(
        a.reshape(rows, feats // 2, 2).astype(bf16), i32)


def _proj_body(x_ref, wl_ref, wr_ref, outlf_ref, outrf_ref, outlc_ref):
    x = x_ref[...]
    xl = jnp.dot(x, wl_ref[...], preferred_element_type=f32)
    xr = jnp.dot(x, wr_ref[...], preferred_element_type=f32)
    xlb = xl.astype(bf16)
    outlf_ref[...] = xlb
    outrf_ref[...] = xr.astype(bf16)
    outlc_ref[0] = xlb[:, :128]
    outlc_ref[1] = xlb[:, 128:]


def _proj(x, wl, wr):
    out = pl.pallas_call(
        _proj_body,
        grid=(N // RB,),
        in_specs=[
            pl.BlockSpec((RB, D1), lambda i: (i, 0)),
            pl.BlockSpec((D1, H), lambda i: (0, 0)),
            pl.BlockSpec((D1, H), lambda i: (0, 0)),
        ],
        out_specs=[
            pl.BlockSpec((RB, H), lambda i: (i, 0)),
            pl.BlockSpec((RB, H), lambda i: (i, 0)),
            pl.BlockSpec((2, RB, 128), lambda i: (0, i, 0)),
        ],
        out_shape=[
            jax.ShapeDtypeStruct((N, H), bf16),
            jax.ShapeDtypeStruct((N, H), bf16),
            jax.ShapeDtypeStruct((2, N, 128), bf16),
        ],
    )(x, wl, wr)
    return (_pack_rows(out[0], N, H), _pack_rows(out[1], N, H),
            _pack_rows(out[2].reshape(2 * N, 128), 2 * N, 128))


def _mid_body(acc_ref, b1_ref, wl2_ref, wr2_ref, outl_ref, outr_ref):
    lo = acc_ref[0]
    hi = acc_ref[1]
    den = lo[:, 128:129] + 1e-16
    h = jnp.concatenate([lo[:, :128], hi[:, :128]], axis=1) / den + b1_ref[...]
    h = jnp.maximum(h, 0.0)
    outl_ref[...] = jnp.dot(h, wl2_ref[...],
                            preferred_element_type=f32).astype(bf16)
    outr_ref[...] = jnp.dot(h, wr2_ref[...],
                            preferred_element_type=f32).astype(bf16)


def _mid(acc1, b1, wl2, wr2):
    return pl.pallas_call(
        _mid_body,
        grid=(N // RB,),
        in_specs=[
            pl.BlockSpec((2, RB, 144), lambda i: (0, i, 0)),
            pl.BlockSpec((1, H), lambda i: (0, 0)),
            pl.BlockSpec((H, 2 * L), lambda i: (0, 0)),
            pl.BlockSpec((H, 2 * L), lambda i: (0, 0)),
        ],
        out_specs=[
            pl.BlockSpec((RB, 2 * L), lambda i: (i, 0)),
            pl.BlockSpec((RB, 2 * L), lambda i: (i, 0)),
        ],
        out_shape=[
            jax.ShapeDtypeStruct((N, 2 * L), bf16),
            jax.ShapeDtypeStruct((N, 2 * L), bf16),
        ],
    )(acc1, b1.reshape(1, H), wl2, wr2)


def _fin_body(acc_ref, b2_ref, eps_ref, wm_ref, wlogv_ref, epsw_ref,
              mean_ref, stats_ref):
    a0 = acc_ref[0]
    a1 = acc_ref[1]
    den = a0[:, 64:65] + a1[:, 64:65] + 1e-16
    stats = (a0[:, :64] + a1[:, :64]) / den + b2_ref[...]
    mu = stats[:, :L]
    logvar = stats[:, L:]
    z = mu + eps_ref[...] * jnp.exp(0.5 * logvar)
    wlin = wm_ref[...] + epsw_ref[...] * jnp.exp(0.5 * wlogv_ref[...])
    wmax = jnp.max(wlin, axis=1, keepdims=True)
    we = jnp.exp(wlin - wmax)
    w = we / jnp.sum(we, axis=1, keepdims=True)
    mean_ref[...] = jnp.dot(z, w, preferred_element_type=f32)
    stats_ref[...] = stats


def _fin(acc2, b2, eps, wm, wlogv, epsw):
    D = wm.shape[1]
    return pl.pallas_call(
        _fin_body,
        grid=(N // RB,),
        in_specs=[
            pl.BlockSpec((2, RB, 80), lambda i: (0, i, 0)),
            pl.BlockSpec((1, 2 * L), lambda i: (0, 0)),
            pl.BlockSpec((RB, L), lambda i: (i, 0)),
            pl.BlockSpec((L, D), lambda i: (0, 0)),
            pl.BlockSpec((L, D), lambda i: (0, 0)),
            pl.BlockSpec((L, D), lambda i: (0, 0)),
        ],
        out_specs=[
            pl.BlockSpec((RB, D), lambda i: (i, 0)),
            pl.BlockSpec((RB, 2 * L), lambda i: (i, 0)),
        ],
        out_shape=[
            jax.ShapeDtypeStruct((N, D), f32),
            jax.ShapeDtypeStruct((N, 2 * L), f32),
        ],
    )(acc2, b2.reshape(1, 2 * L), eps, wm, wlogv, epsw)


def _mesh():
    return plsc.VectorSubcoreMesh(core_axis_name="c", subcore_axis_name="s")


def _zero_rows(buf, nrows, wo):
    def zrow(j, carry):
        for k in range(wo // 16):
            buf[j, pl.ds(k * 16, 16)] = jnp.zeros((16,), f32)
        return carry

    lax.fori_loop(0, nrows, zrow, 0)


def _zero_acc(acc_sh, zbuf, s):
    for k in range(RPT // BE):
        pltpu.sync_copy(zbuf, acc_sh.at[pl.ds(s * RPT + k * BE, BE)])
    rem = RPT % BE
    if rem:
        pltpu.sync_copy(zbuf.at[pl.ds(0, rem)],
                        acc_sh.at[pl.ds(s * RPT + RPT - rem, rem)])


def _writeout(acc_sh, out_hbm, c, s, wo):
    for k in range(RPT // BE):
        r0 = s * RPT + k * BE
        pltpu.sync_copy(acc_sh.at[pl.ds(r0, BE)],
                        out_hbm.at[c, pl.ds(r0, BE)])
    rem = RPT % BE
    if rem:
        r0 = s * RPT + RPT - rem
        pltpu.sync_copy(acc_sh.at[pl.ds(r0, rem)],
                        out_hbm.at[c, pl.ds(r0, rem)])


def _make_score1():
    ept = E // NW
    nch = ept // BE

    @functools.partial(
        pl.kernel,
        compiler_params=_SC_PARAMS,
        out_type=jax.ShapeDtypeStruct((E,), f32),
        mesh=_mesh(),
        scratch_types=[
            pltpu.VMEM((ept,), i32),
            pltpu.VMEM((ept,), i32),
            pltpu.VMEM((BE, HP), i32),
            pltpu.VMEM((BE, HP), i32),
            pltpu.VMEM((BE, HP), i32),
            pltpu.VMEM((BE, HP), i32),
            pltpu.VMEM((ept,), f32),
            pltpu.VMEM((H, 16), f32),
            pltpu.SemaphoreType.DMA,
            pltpu.SemaphoreType.DMA,
        ],
    )
    def score1(xl_hbm, xr_hbm, src_hbm, dst_hbm, att_hbm, ex_hbm,
               srcb, dstb, bl0, br0, bl1, br1, exb, attv, sem0, sem1):
        wid = lax.axis_index("s") * NC + lax.axis_index("c")
        base = wid * ept
        pltpu.sync_copy(att_hbm, attv)
        pltpu.sync_copy(src_hbm.at[pl.ds(base, ept)], srcb)
        pltpu.sync_copy(dst_hbm.at[pl.ds(base, ept)], dstb)
        it16 = _iota16()
        sets = ((bl0, br0, sem0), (bl1, br1, sem1))

        def issue(ci, bl, br, sem):
            sl = pl.ds(ci * BE, BE)
            pltpu.async_copy(xl_hbm.at[srcb.at[sl]], bl, sem)
            pltpu.async_copy(xr_hbm.at[dstb.at[sl]], br, sem)

        def drain(ci, bl, br, sem):
            sl = pl.ds(ci * BE, BE)
            pltpu.make_async_copy(xl_hbm.at[srcb.at[sl]], bl, sem).wait()
            pltpu.make_async_copy(xr_hbm.at[dstb.at[sl]], br, sem).wait()

        def compute(ci, bl, br):
            for g in range(BE // 16):
                rows = it16 + g * 16

                def hbody(hh, acc):
                    col = jnp.zeros((16,), i32) + hh
                    pwl = plsc.load_gather(bl, [rows, col])
                    pwr = plsc.load_gather(br, [rows, col])
                    l0, l1 = _unpack2(pwl)
                    r0, r1 = _unpack2(pwr)
                    u0 = l0 + r0
                    u0 = jnp.maximum(u0, 0.2 * u0)
                    u1 = l1 + r1
                    u1 = jnp.maximum(u1, 0.2 * u1)
                    return acc + u0 * attv[2 * hh] + u1 * attv[2 * hh + 1]

                acc = lax.fori_loop(0, HP, hbody, jnp.zeros((16,), f32),
                                    unroll=8)
                exb[pl.ds(ci * BE + g * 16, 16)] = jnp.exp(acc)

        issue(0, *sets[0])
        issue(1, *sets[1])

        def step(i2, carry):
            for par in range(2):
                ci = 2 * i2 + par
                drain(ci, *sets[par])
                compute(ci, *sets[par][:2])

                @pl.when(ci + 2 < nch)
                def _():
                    issue(ci + 2, *sets[par])
            return carry

        lax.fori_loop(0, (nch - 1) // 2, step, 0)
        ci = nch - 1
        drain(ci, *sets[0])
        compute(ci, *sets[0][:2])
        pltpu.sync_copy(exb, ex_hbm.at[pl.ds(base, ept)])

    return score1


def _make_accum1():
    wf, wo = 128, 144
    ept = E // NS
    SB = 10 * BE
    nblk = ept // SB

    @functools.partial(
        pl.kernel,
        compiler_params=_SC_PARAMS,
        out_type=jax.ShapeDtypeStruct((2, N, wo), f32),
        mesh=_mesh(),
        scratch_types=[
            pltpu.VMEM((SB,), i32),
            pltpu.VMEM((SB,), i32),
            pltpu.VMEM((SB,), f32),
            pltpu.VMEM((BE, wf // 2), i32),
            pltpu.VMEM((BE, wf // 2), i32),
            pltpu.VMEM((BE, wo), f32),
            pltpu.VMEM((BE, wo), f32),
            pltpu.VMEM((BE,), i32),
            pltpu.VMEM((BE,), i32),
            pltpu.VMEM_SHARED((N, wo), f32),
            pltpu.SemaphoreType.DMA,
            pltpu.SemaphoreType.DMA,
            pltpu.SemaphoreType.DMA,
            pltpu.SemaphoreType.DMA,
        ],
    )
    def accum1(xl_hbm, src_hbm, dst_hbm, ex_hbm, out_hbm,
               srcb, dstb, exb, rows0, rows1, scal0, scal1, dstv0, dstv1,
               acc_sh, sem0, sem1, ssem0, ssem1):
        c = lax.axis_index("c")
        s = lax.axis_index("s")
        base = s * ept
        it16 = _iota16()
        _zero_rows(scal0, BE, wo)
        _zero_rows(scal1, BE, wo)
        _zero_acc(acc_sh, scal0, s)
        plsc.subcore_barrier()

        sems = (sem0, sem1)
        rows_sets = (rows0, rows1)
        scals = (scal0, scal1)
        dstvs = (dstv0, dstv1)
        ssems = (ssem0, ssem1)

        def issue(ci, rows, sem):
            pltpu.async_copy(xl_hbm.at[srcb.at[pl.ds(ci * BE, BE)]],
                             rows, sem)

        def drain(ci, rows, sem):
            pltpu.make_async_copy(xl_hbm.at[srcb.at[pl.ds(ci * BE, BE)]],
                                  rows, sem).wait()

        def compute(gi, ci, rows, par):
            scal = scals[par]
            dstv = dstvs[par]
            ssem = ssems[par]

            @pl.when(gi >= 2)
            def _():
                pltpu.make_async_copy(scal, acc_sh.at[dstv], ssem).wait()

            for g in range(BE // 16):
                r16 = it16 + g * 16
                sl = pl.ds(ci * BE + g * 16, 16)
                exg = exb[sl]
                dstv[pl.ds(g * 16, 16)] = dstb[sl]
                plsc.store_scatter(scal, [r16, jnp.zeros((16,), i32) + wf],
                                   exg)

                def hbody(hh, carry2):
                    col = jnp.zeros((16,), i32) + hh
                    w = plsc.load_gather(rows, [r16, col])
                    v0, v1 = _unpack2(w)
                    plsc.store_scatter(scal, [r16, 2 * col], v0 * exg)
                    plsc.store_scatter(scal, [r16, 2 * col + 1], v1 * exg)
                    return carry2

                lax.fori_loop(0, wf // 2, hbody, 0, unroll=8)
            pltpu.async_copy(scal, acc_sh.at[dstv], ssem, add=True)

        def block(bi, carry):
            bo = base + bi * SB
            pltpu.async_copy(src_hbm.at[pl.ds(bo, SB)], srcb, sem0)
            pltpu.async_copy(dst_hbm.at[pl.ds(bo, SB)], dstb, sem0)
            pltpu.async_copy(ex_hbm.at[pl.ds(bo, SB)], exb, sem0)
            pltpu.make_async_copy(src_hbm.at[pl.ds(bo, SB)], srcb, sem0).wait()
            pltpu.make_async_copy(dst_hbm.at[pl.ds(bo, SB)], dstb, sem0).wait()
            pltpu.make_async_copy(ex_hbm.at[pl.ds(bo, SB)], exb, sem0).wait()

            def adj(j, carry2):
                sl = pl.ds(j * 16, 16)
                srcb[sl] = srcb[sl] + c * N
                return carry2

            lax.fori_loop(0, SB // 16, adj, 0)
            issue(0, rows0, sem0)
            issue(1, rows1, sem1)

            def step(i2, carry2):
                for par in range(2):
                    ci = 2 * i2 + par
                    drain(ci, rows_sets[par], sems[par])
                    compute(bi * (SB // BE) + ci, ci, rows_sets[par], par)

                    @pl.when(ci + 2 < SB // BE)
                    def _():
                        issue(ci + 2, rows_sets[par], sems[par])
                return carry2

            lax.fori_loop(0, SB // BE // 2, step, 0)
            return carry

        lax.fori_loop(0, nblk, block, 0)
        pltpu.make_async_copy(scal0, acc_sh.at[dstv0], ssem0).wait()
        pltpu.make_async_copy(scal1, acc_sh.at[dstv1], ssem1).wait()
        plsc.subcore_barrier()
        _writeout(acc_sh, out_hbm, c, s, wo)

    return accum1


def _make_gat2():
    wf, wo = 64, 80
    ept = E // NW
    nch = ept // BE

    @functools.partial(
        pl.kernel,
        compiler_params=_SC_PARAMS,
        out_type=jax.ShapeDtypeStruct((2, N, wo), f32),
        mesh=_mesh(),
        scratch_types=[
            pltpu.VMEM((ept,), i32),
            pltpu.VMEM((ept,), i32),
            pltpu.VMEM((BE, wf // 2), i32),
            pltpu.VMEM((BE, wf // 2), i32),
            pltpu.VMEM((BE, wf // 2), i32),
            pltpu.VMEM((BE, wf // 2), i32),
            pltpu.VMEM((BE, wo), f32),
            pltpu.VMEM((BE, wo), f32),
            pltpu.VMEM((wf, 16), f32),
            pltpu.VMEM((BE,), i32),
            pltpu.VMEM((BE,), i32),
            pltpu.VMEM_SHARED((N, wo), f32),
            pltpu.SemaphoreType.DMA,
            pltpu.SemaphoreType.DMA,
            pltpu.SemaphoreType.DMA,
            pltpu.SemaphoreType.DMA,
        ],
    )
    def gat2(xl_hbm, xr_hbm, src_hbm, dst_hbm, att_hbm, out_hbm,
             srcb, dstb, bl0, br0, bl1, br1, scal0, scal1, attv,
             dstv0, dstv1, acc_sh, sem0, sem1, ssem0, ssem1):
        c = lax.axis_index("c")
        s = lax.axis_index("s")
        wid = s * NC + c
        base = wid * ept
        pltpu.sync_copy(att_hbm, attv)
        pltpu.sync_copy(src_hbm.at[pl.ds(base, ept)], srcb)
        pltpu.sync_copy(dst_hbm.at[pl.ds(base, ept)], dstb)
        it16 = _iota16()
        _zero_rows(scal0, BE, wo)
        _zero_rows(scal1, BE, wo)
        _zero_acc(acc_sh, scal0, s)
        plsc.subcore_barrier()

        sets = ((bl0, br0, scal0, dstv0, sem0, ssem0),
                (bl1, br1, scal1, dstv1, sem1, ssem1))

        def issue(ci, bl, br, scal, dstv, sem, ssem):
            sl = pl.ds(ci * BE, BE)
            pltpu.async_copy(xl_hbm.at[srcb.at[sl]], bl, sem)
            pltpu.async_copy(xr_hbm.at[dstb.at[sl]], br, sem)

        def drain(ci, bl, br, scal, dstv, sem, ssem):
            sl = pl.ds(ci * BE, BE)
            pltpu.make_async_copy(xl_hbm.at[srcb.at[sl]], bl, sem).wait()
            pltpu.make_async_copy(xr_hbm.at[dstb.at[sl]], br, sem).wait()

        def compute(ci, bl, br, scal, dstv, sem, ssem):
            @pl.when(ci >= 2)
            def _():
                pltpu.make_async_copy(scal, acc_sh.at[dstv], ssem).wait()

            for g in range(BE // 16):
                r16 = it16 + g * 16
                dstv[pl.ds(g * 16, 16)] = dstb[pl.ds(ci * BE + g * 16, 16)]

                def hbody(hh, acc):
                    col = jnp.zeros((16,), i32) + hh
                    pwl = plsc.load_gather(bl, [r16, col])
                    pwr = plsc.load_gather(br, [r16, col])
                    l0, l1 = _unpack2(pwl)
                    r0, r1 = _unpack2(pwr)
                    u0 = l0 + r0
                    u0 = jnp.maximum(u0, 0.2 * u0)
                    u1 = l1 + r1
                    u1 = jnp.maximum(u1, 0.2 * u1)
                    return acc + u0 * attv[2 * hh] + u1 * attv[2 * hh + 1]

                acc = lax.fori_loop(0, wf // 2, hbody, jnp.zeros((16,), f32),
                                    unroll=8)
                exg = jnp.exp(acc)
                plsc.store_scatter(scal, [r16, jnp.zeros((16,), i32) + wf],
                                   exg)

                def sbody(hh, carry2):
                    col = jnp.zeros((16,), i32) + hh
                    w = plsc.load_gather(bl, [r16, col])
                    v0, v1 = _unpack2(w)
                    plsc.store_scatter(scal, [r16, 2 * col], v0 * exg)
                    plsc.store_scatter(scal, [r16, 2 * col + 1], v1 * exg)
                    return carry2

                lax.fori_loop(0, wf // 2, sbody, 0, unroll=8)
            pltpu.async_copy(scal, acc_sh.at[dstv], ssem, add=True)

        issue(0, *sets[0])
        issue(1, *sets[1])

        def step(i2, carry):
            for par in range(2):
                ci = 2 * i2 + par
                drain(ci, *sets[par])
                compute(ci, *sets[par])

                @pl.when(ci + 2 < nch)
                def _():
                    issue(ci + 2, *sets[par])
            return carry

        lax.fori_loop(0, (nch - 1) // 2, step, 0)
        ci = nch - 1
        drain(ci, *sets[0])
        compute(ci, *sets[0])
        pltpu.make_async_copy(scal0, acc_sh.at[dstv0], ssem0).wait()
        pltpu.make_async_copy(scal1, acc_sh.at[dstv1], ssem1).wait()
        plsc.subcore_barrier()
        _writeout(acc_sh, out_hbm, c, s, wo)

    return gat2


def kernel(x1, x2, edge_index,
           enc1_Wl1, enc1_Wr1, enc1_a1, enc1_b1,
           enc1_Wl2, enc1_Wr2, enc1_a2, enc1_b2,
           enc2_Wl1, enc2_Wr1, enc2_a1, enc2_b1,
           enc2_Wl2, enc2_Wr2, enc2_a2, enc2_b2,
           dec1_w_m, dec1_w_logv, dec2_w_m, dec2_w_logv):
    src, dst = edge_index[0], edge_index[1]
    rk = jax.random.key(1)
    eps1 = jax.random.normal(jax.random.fold_in(rk, 0), (N, L), f32)
    eps2 = jax.random.normal(jax.random.fold_in(rk, 1), (N, L), f32)
    epsw1 = jax.random.normal(jax.random.fold_in(rk, 2), (L, D1), f32)
    epsw2 = jax.random.normal(jax.random.fold_in(rk, 3), (L, D2), f32)

    score1 = _make_score1()
    accum1 = _make_accum1()
    gat2 = _make_gat2()

    def encoder(x, wl1, wr1, a1, b1, wl2, wr2, a2, b2):
        xlf, xrf, xlc = _proj(x, wl1, wr1)
        a1sp = jnp.broadcast_to(a1[:, None], (H, 16))
        a2sp = jnp.broadcast_to(a2[:, None], (2 * L, 16))
        ex1 = score1(xlf, xrf, src, dst, a1sp)
        acc1 = accum1(xlc, src, dst, ex1)
        xl2, xr2 = _mid(acc1, b1, wl2, wr2)
        xl2p = _pack_rows(xl2, N, 2 * L)
        xr2p = _pack_rows(xr2, N, 2 * L)
        acc2 = gat2(xl2p, xr2p, src, dst, a2sp)
        return acc2

    acc2_1 = encoder(x1, enc1_Wl1, enc1_Wr1, enc1_a1, enc1_b1,
                     enc1_Wl2, enc1_Wr2, enc1_a2, enc1_b2)
    acc2_2 = encoder(x2, enc2_Wl1, enc2_Wr1, enc2_a1, enc2_b1,
                     enc2_Wl2, enc2_Wr2, enc2_a2, enc2_b2)

    mean1, stats1 = _fin(acc2_1, enc1_b2, eps1, dec1_w_m, dec1_w_logv, epsw1)
    mean2, stats2 = _fin(acc2_2, enc2_b2, eps2, dec2_w_m, dec2_w_logv, epsw2)

    return (mean1, mean2, stats1[:, :L], stats1[:, L:],
            stats2[:, :L], stats2[:, L:])
